# baseline (speedup 1.0000x reference)
import functools

import jax
import jax.numpy as jnp
from jax import lax
from jax.experimental import pallas as pl
from jax.experimental.pallas import tpu as pltpu
from jax.experimental.pallas import tpu_sc as plsc

N = 10000
NPAD = 10240
SLAB = 640
D = 128
DH = 64
E = 320000
ECHUNK = 128
CPT = 160
EPAD = CPT * ECHUNK * 16
K_ITERS = 10
ALPHA = 0.1
BN_EPS = 1e-5
SUP = 16
PCH = 32


def _mlp_body(x_ref, w1_ref, b1_ref, g_ref, bt_ref, w2_ref, b2_ref, o_ref):
    x = x_ref[...]
    h1 = lax.dot_general(x, w1_ref[...], (((1,), (1,)), ((), ())),
                         preferred_element_type=jnp.float32)
    h1 = h1 + b1_ref[...]
    mean = jnp.mean(h1, axis=0, keepdims=True)
    hc = h1 - mean
    var = jnp.mean(hc * hc, axis=0, keepdims=True)
    s = g_ref[...] * lax.rsqrt(var + BN_EPS)
    h = hc * s + bt_ref[...]
    h2 = lax.dot_general(h, w2_ref[...], (((1,), (1,)), ((), ())),
                         preferred_element_type=jnp.float32)
    h2 = h2 + b2_ref[...]
    o_ref[0, pl.ds(0, N), :] = h2[:, 0:DH]
    o_ref[1, pl.ds(0, N), :] = h2[:, DH:D]
    pad = jnp.zeros((NPAD - N, DH), jnp.float32)
    o_ref[0, pl.ds(N, NPAD - N), :] = pad
    o_ref[1, pl.ds(N, NPAD - N), :] = pad


def _lsm_body(p_ref, out_ref, emb_ref):
    a = p_ref[0, pl.ds(0, N), :]
    b = p_ref[1, pl.ds(0, N), :]
    emb = jnp.concatenate([a, b], axis=1)
    m = jnp.max(emb, axis=1, keepdims=True)
    e = emb - m
    lse = jnp.log(jnp.sum(jnp.exp(e), axis=1, keepdims=True))
    out_ref[...] = e - lse
    emb_ref[...] = emb


_mesh = plsc.VectorSubcoreMesh(core_axis_name="c", subcore_axis_name="s",
                               num_cores=2, num_subcores=16)


@functools.partial(
    pl.kernel,
    out_type=jax.ShapeDtypeStruct((2, NPAD, DH), jnp.float32),
    mesh=_mesh,
    compiler_params=pltpu.CompilerParams(use_tc_tiling_on_sc=False),
    scratch_types=[
        pltpu.VMEM_SHARED((NPAD, DH), jnp.float32),
        pltpu.VMEM_SHARED((NPAD, DH), jnp.float32),
        pltpu.VMEM_SHARED((NPAD,), jnp.float32),
        pltpu.VMEM((SUP * ECHUNK,), jnp.int32),
        pltpu.VMEM((SUP * ECHUNK,), jnp.int32),
        pltpu.VMEM((SUP * ECHUNK,), jnp.int32),
        pltpu.VMEM((SUP * ECHUNK,), jnp.int32),
        pltpu.VMEM((ECHUNK, DH), jnp.float32),
        pltpu.VMEM((ECHUNK, DH), jnp.float32),
        pltpu.VMEM((PCH, DH), jnp.float32),
        pltpu.VMEM((PCH, DH), jnp.float32),
        pltpu.VMEM((PCH, DH), jnp.float32),
        pltpu.VMEM((16, DH), jnp.float32),
        pltpu.VMEM((SLAB + 16,), jnp.float32),
        pltpu.VMEM((ECHUNK,), jnp.float32),
        pltpu.SemaphoreType.DMA,
        pltpu.SemaphoreType.DMA,
        pltpu.SemaphoreType.DMA,
        pltpu.SemaphoreType.DMA,
        pltpu.SemaphoreType.DMA,
        pltpu.SemaphoreType.DMA,
        pltpu.VMEM((PCH, DH), jnp.float32),
        pltpu.VMEM((PCH, DH), jnp.float32),
        pltpu.VMEM((PCH, DH), jnp.float32),
        pltpu.SemaphoreType.DMA,
        pltpu.SemaphoreType.DMA,
        pltpu.SemaphoreType.DMA,
        pltpu.SemaphoreType.DMA,
        pltpu.SemaphoreType.DMA,
        pltpu.SemaphoreType.DMA,
    ],
)
def _propagate(h0_hbm, src_hbm, dst_hbm, out_hbm, h_s, agg_s, deg_s,
               srcbuf0, srcbuf1, dstbuf0, dstbuf1, rows0, rows1,
               h0buf, aggbuf, hscbuf, zerobuf, disbuf, onesbuf,
               idx_sem0, idx_sem1, gsem0, gsem1, ssem0, ssem1,
               h0buf1, aggbuf1, hscbuf1, asem0, asem1, hsem0, hsem1,
               osem0, osem1):
    c = lax.axis_index("c")
    t = lax.axis_index("s")
    slab = t * SLAB

    sbufs = (srcbuf0, srcbuf1)
    dbufs = (dstbuf0, dstbuf1)
    isems = (idx_sem0, idx_sem1)
    rbufs = (rows0, rows1)
    gsems = (gsem0, gsem1)
    ssems = (ssem0, ssem1)
    NSUP = CPT // SUP
    NPCH = SLAB // PCH

    def issue_idx(s_dyn, par):
        base_e = (t * CPT + s_dyn * SUP) * ECHUNK
        pltpu.async_copy(src_hbm.at[pl.ds(base_e, SUP * ECHUNK)],
                         sbufs[par], isems[par])
        pltpu.async_copy(dst_hbm.at[pl.ds(base_e, SUP * ECHUNK)],
                         dbufs[par], isems[par])

    def wait_idx(par):
        pltpu.make_async_copy(src_hbm.at[pl.ds(0, SUP * ECHUNK)],
                              sbufs[par], isems[par]).wait()
        pltpu.make_async_copy(dst_hbm.at[pl.ds(0, SUP * ECHUNK)],
                              dbufs[par], isems[par]).wait()

    pwa = (aggbuf, aggbuf1)
    pwh = (hscbuf, hscbuf1)
    pwo = (h0buf, h0buf1)
    asems = (asem0, asem1)
    hsems = (hsem0, hsem1)
    osems = (osem0, osem1)

    def issue_pw(ch_dyn, par):
        gb = slab + ch_dyn * PCH
        pltpu.async_copy(agg_s.at[pl.ds(gb, PCH), :], pwa[par], asems[par])
        pltpu.async_copy(h_s.at[pl.ds(gb, PCH), :], pwh[par], hsems[par])
        pltpu.async_copy(h0_hbm.at[c, pl.ds(gb, PCH), :], pwo[par],
                         osems[par])

    def wait_pw(par):
        dummy = h0_hbm.at[c, pl.ds(0, PCH), :]
        pltpu.make_async_copy(dummy, pwa[par], asems[par]).wait()
        pltpu.make_async_copy(dummy, pwh[par], hsems[par]).wait()
        pltpu.make_async_copy(dummy, pwo[par], osems[par]).wait()

    def half(par):
        sb, db = sbufs[par], dbufs[par]
        gd = [None] * SUP
        sd = [None] * SUP
        gd[0] = pltpu.async_copy(h_s.at[sb.at[pl.ds(0, ECHUNK)]],
                                 rbufs[0], gsems[0])
        for j in range(SUP):
            gd[j].wait()
            sd[j] = pltpu.async_copy(
                rbufs[j % 2], agg_s.at[db.at[pl.ds(j * ECHUNK, ECHUNK)]],
                ssems[j % 2], add=True)
            if j + 1 < SUP:
                if j >= 1:
                    sd[j - 1].wait()
                gd[j + 1] = pltpu.async_copy(
                    h_s.at[sb.at[pl.ds((j + 1) * ECHUNK, ECHUNK)]],
                    rbufs[(j + 1) % 2], gsems[(j + 1) % 2])
        sd[SUP - 2].wait()
        sd[SUP - 1].wait()

    @pl.loop(0, ECHUNK // 16)
    def _(v):
        onesbuf[pl.ds(v * 16, 16)] = jnp.ones((16,), jnp.float32)

    @pl.loop(0, 16)
    def _(r):
        for kk in range(DH // 16):
            zerobuf[r, pl.ds(kk * 16, 16)] = jnp.zeros((16,), jnp.float32)

    @pl.loop(0, SLAB // ECHUNK)
    def _(p):
        pltpu.sync_copy(onesbuf, deg_s.at[pl.ds(slab + p * ECHUNK, ECHUNK)])

    plsc.subcore_barrier()

    def dhalf(par):
        db = dbufs[par]
        sd = [None] * SUP
        for j in range(SUP):
            if j >= 2:
                sd[j - 2].wait()
            sd[j] = pltpu.async_copy(
                onesbuf, deg_s.at[db.at[pl.ds(j * ECHUNK, ECHUNK)]],
                ssems[j % 2], add=True)
        sd[SUP - 2].wait()
        sd[SUP - 1].wait()

    issue_idx(0, 0)

    @pl.loop(0, NSUP // 2)
    def _(gi):
        s0 = 2 * gi
        wait_idx(0)
        issue_idx(s0 + 1, 1)
        dhalf(0)
        wait_idx(1)

        @pl.when(s0 + 2 < NSUP)
        def _():
            issue_idx(s0 + 2, 0)

        dhalf(1)

    plsc.subcore_barrier()

    pltpu.sync_copy(deg_s.at[pl.ds(slab, SLAB)], disbuf.at[pl.ds(0, SLAB)])

    @pl.loop(0, SLAB // 16)
    def _(v):
        d = disbuf[pl.ds(v * 16, 16)]
        i = lax.bitcast_convert_type(d, jnp.int32)
        i = jnp.int32(0x5F3759DF) - (i >> 1)
        y = lax.bitcast_convert_type(i, jnp.float32)
        for _ in range(4):
            y = y * (1.5 - 0.5 * d * y * y)
        disbuf[pl.ds(v * 16, 16)] = y

    @pl.loop(0, SLAB // PCH)
    def _(ch):
        base_l = ch * PCH
        pltpu.sync_copy(h0_hbm.at[c, pl.ds(slab + base_l, PCH), :], h0buf)

        @pl.loop(0, PCH)
        def _(i):
            li = base_l + i
            dv = disbuf[pl.ds(li, 16)]
            dspl = jnp.full((16,), dv[0], jnp.float32)
            for kk in range(DH // 16):
                sl = pl.ds(kk * 16, 16)
                hscbuf[i, sl] = h0buf[i, sl] * dspl

        pltpu.sync_copy(hscbuf, h_s.at[pl.ds(slab + base_l, PCH), :])
        for z in range(PCH // 16):
            pltpu.sync_copy(zerobuf,
                            agg_s.at[pl.ds(slab + base_l + z * 16, 16), :])

    plsc.subcore_barrier()

    issue_idx(0, 0)

    @pl.loop(0, K_ITERS)
    def _(k):
        @pl.loop(0, NSUP // 2)
        def _(gi):
            s0 = 2 * gi
            wait_idx(0)
            issue_idx(s0 + 1, 1)
            half(0)
            wait_idx(1)

            @pl.when(s0 + 2 < NSUP)
            def _():
                issue_idx(s0 + 2, 0)

            half(1)

        @pl.when(k < K_ITERS - 1)
        def _():
            issue_idx(0, 0)

        plsc.subcore_barrier()

        issue_pw(0, 0)

        @pl.loop(0, NPCH // 2)
        def _(pp):
            for par in range(2):
                ch = 2 * pp + par
                base_l = ch * PCH
                gbase = slab + base_l
                wait_pw(par)
                if par == 0:
                    issue_pw(ch + 1, 1)
                else:
                    @pl.when(ch + 1 < NPCH)
                    def _():
                        issue_pw(ch + 1, 0)
                ab, hb, ob = pwa[par], pwh[par], pwo[par]
                for z in range(PCH // 16):
                    pltpu.sync_copy(zerobuf,
                                    agg_s.at[pl.ds(gbase + z * 16, 16), :])

                @pl.loop(0, PCH)
                def _(i):
                    li = base_l + i
                    dv = disbuf[pl.ds(li, 16)]
                    dspl = jnp.full((16,), dv[0], jnp.float32)
                    d1 = dspl * (1.0 - ALPHA)
                    for kk in range(DH // 16):
                        sl = pl.ds(kk * 16, 16)
                        hn = (d1 * (ab[i, sl] + hb[i, sl])
                              + ALPHA * ob[i, sl])
                        ab[i, sl] = hn
                        hb[i, sl] = hn * dspl

                @pl.when(k < K_ITERS - 1)
                def _():
                    pltpu.sync_copy(hb, h_s.at[pl.ds(gbase, PCH), :])

                @pl.when(k == K_ITERS - 1)
                def _():
                    pltpu.sync_copy(ab, out_hbm.at[c, pl.ds(gbase, PCH), :])

        plsc.subcore_barrier()


def kernel(x, edge_index, W1, b1, gamma, beta, W2, b2):
    h0p = pl.pallas_call(
        _mlp_body,
        out_shape=jax.ShapeDtypeStruct((2, NPAD, DH), jnp.float32),
    )(x, W1, b1.reshape(1, D), gamma.reshape(1, D), beta.reshape(1, D),
      W2, b2.reshape(1, D))

    src = edge_index[0].astype(jnp.int32)
    dst = edge_index[1].astype(jnp.int32)
    pad = N + (jnp.arange(EPAD - E, dtype=jnp.int32) % (NPAD - N))
    src2 = jnp.concatenate([src, pad])
    dst2 = jnp.concatenate([dst, pad])

    embp = _propagate(h0p, src2, dst2)

    out, emb = pl.pallas_call(
        _lsm_body,
        out_shape=[jax.ShapeDtypeStruct((N, D), jnp.float32),
                   jax.ShapeDtypeStruct((N, D), jnp.float32)],
    )(embp)
    return (out, emb)


# speedup vs baseline: 1.0103x; 1.0103x over previous
import functools

import jax
import jax.numpy as jnp
from jax import lax
from jax.experimental import pallas as pl
from jax.experimental.pallas import tpu as pltpu
from jax.experimental.pallas import tpu_sc as plsc

N = 10000
NPAD = 10240
SLAB = 640
D = 128
DH = 64
E = 320000
ECHUNK = 128
CPT = 160
EPAD = CPT * ECHUNK * 16
K_ITERS = 10
ALPHA = 0.1
BN_EPS = 1e-5
SUP = 20
PCH = 32


def _mlp_body(x_ref, w1_ref, b1_ref, g_ref, bt_ref, w2_ref, b2_ref, o_ref):
    x = x_ref[...]
    h1 = lax.dot_general(x, w1_ref[...], (((1,), (1,)), ((), ())),
                         preferred_element_type=jnp.float32)
    h1 = h1 + b1_ref[...]
    mean = jnp.mean(h1, axis=0, keepdims=True)
    hc = h1 - mean
    var = jnp.mean(hc * hc, axis=0, keepdims=True)
    s = g_ref[...] * lax.rsqrt(var + BN_EPS)
    h = hc * s + bt_ref[...]
    h2 = lax.dot_general(h, w2_ref[...], (((1,), (1,)), ((), ())),
                         preferred_element_type=jnp.float32)
    h2 = h2 + b2_ref[...]
    o_ref[0, pl.ds(0, N), :] = h2[:, 0:DH]
    o_ref[1, pl.ds(0, N), :] = h2[:, DH:D]
    pad = jnp.zeros((NPAD - N, DH), jnp.float32)
    o_ref[0, pl.ds(N, NPAD - N), :] = pad
    o_ref[1, pl.ds(N, NPAD - N), :] = pad


def _lsm_body(p_ref, out_ref, emb_ref):
    a = p_ref[0, pl.ds(0, N), :]
    b = p_ref[1, pl.ds(0, N), :]
    emb = jnp.concatenate([a, b], axis=1)
    m = jnp.max(emb, axis=1, keepdims=True)
    e = emb - m
    lse = jnp.log(jnp.sum(jnp.exp(e), axis=1, keepdims=True))
    out_ref[...] = e - lse
    emb_ref[...] = emb


_mesh = plsc.VectorSubcoreMesh(core_axis_name="c", subcore_axis_name="s",
                               num_cores=2, num_subcores=16)


@functools.partial(
    pl.kernel,
    out_type=jax.ShapeDtypeStruct((2, NPAD, DH), jnp.float32),
    mesh=_mesh,
    compiler_params=pltpu.CompilerParams(use_tc_tiling_on_sc=False),
    scratch_types=[
        pltpu.VMEM_SHARED((NPAD, DH), jnp.float32),
        pltpu.VMEM_SHARED((NPAD, DH), jnp.float32),
        pltpu.VMEM_SHARED((NPAD,), jnp.float32),
        pltpu.VMEM((SUP * ECHUNK,), jnp.int32),
        pltpu.VMEM((SUP * ECHUNK,), jnp.int32),
        pltpu.VMEM((SUP * ECHUNK,), jnp.int32),
        pltpu.VMEM((SUP * ECHUNK,), jnp.int32),
        pltpu.VMEM((ECHUNK, DH), jnp.float32),
        pltpu.VMEM((ECHUNK, DH), jnp.float32),
        pltpu.VMEM((PCH, DH), jnp.float32),
        pltpu.VMEM((PCH, DH), jnp.float32),
        pltpu.VMEM((PCH, DH), jnp.float32),
        pltpu.VMEM((16, DH), jnp.float32),
        pltpu.VMEM((SLAB + 16,), jnp.float32),
        pltpu.VMEM((ECHUNK,), jnp.float32),
        pltpu.SemaphoreType.DMA,
        pltpu.SemaphoreType.DMA,
        pltpu.SemaphoreType.DMA,
        pltpu.SemaphoreType.DMA,
        pltpu.SemaphoreType.DMA,
        pltpu.SemaphoreType.DMA,
        pltpu.VMEM((PCH, DH), jnp.float32),
        pltpu.VMEM((PCH, DH), jnp.float32),
        pltpu.VMEM((PCH, DH), jnp.float32),
        pltpu.SemaphoreType.DMA,
        pltpu.SemaphoreType.DMA,
        pltpu.SemaphoreType.DMA,
        pltpu.SemaphoreType.DMA,
        pltpu.SemaphoreType.DMA,
        pltpu.SemaphoreType.DMA,
    ],
)
def _propagate(h0_hbm, src_hbm, dst_hbm, out_hbm, h_s, agg_s, deg_s,
               srcbuf0, srcbuf1, dstbuf0, dstbuf1, rows0, rows1,
               h0buf, aggbuf, hscbuf, zerobuf, disbuf, onesbuf,
               idx_sem0, idx_sem1, gsem0, gsem1, ssem0, ssem1,
               h0buf1, aggbuf1, hscbuf1, asem0, asem1, hsem0, hsem1,
               osem0, osem1):
    c = lax.axis_index("c")
    t = lax.axis_index("s")
    slab = t * SLAB

    sbufs = (srcbuf0, srcbuf1)
    dbufs = (dstbuf0, dstbuf1)
    isems = (idx_sem0, idx_sem1)
    rbufs = (rows0, rows1)
    gsems = (gsem0, gsem1)
    ssems = (ssem0, ssem1)
    NSUP = CPT // SUP
    NPCH = SLAB // PCH

    def issue_idx(s_dyn, par):
        base_e = (t * CPT + s_dyn * SUP) * ECHUNK
        pltpu.async_copy(src_hbm.at[pl.ds(base_e, SUP * ECHUNK)],
                         sbufs[par], isems[par])
        pltpu.async_copy(dst_hbm.at[pl.ds(base_e, SUP * ECHUNK)],
                         dbufs[par], isems[par])

    def wait_idx(par):
        pltpu.make_async_copy(src_hbm.at[pl.ds(0, SUP * ECHUNK)],
                              sbufs[par], isems[par]).wait()
        pltpu.make_async_copy(dst_hbm.at[pl.ds(0, SUP * ECHUNK)],
                              dbufs[par], isems[par]).wait()

    pwa = (aggbuf, aggbuf1)
    pwh = (hscbuf, hscbuf1)
    pwo = (h0buf, h0buf1)
    asems = (asem0, asem1)
    hsems = (hsem0, hsem1)
    osems = (osem0, osem1)

    def issue_pw(ch_dyn, par):
        gb = slab + ch_dyn * PCH
        pltpu.async_copy(agg_s.at[pl.ds(gb, PCH), :], pwa[par], asems[par])
        pltpu.async_copy(h_s.at[pl.ds(gb, PCH), :], pwh[par], hsems[par])
        pltpu.async_copy(h0_hbm.at[c, pl.ds(gb, PCH), :], pwo[par],
                         osems[par])

    def wait_pw(par):
        dummy = h0_hbm.at[c, pl.ds(0, PCH), :]
        pltpu.make_async_copy(dummy, pwa[par], asems[par]).wait()
        pltpu.make_async_copy(dummy, pwh[par], hsems[par]).wait()
        pltpu.make_async_copy(dummy, pwo[par], osems[par]).wait()

    def half(par):
        sb, db = sbufs[par], dbufs[par]
        gd = [None] * SUP
        sd = [None] * SUP
        gd[0] = pltpu.async_copy(h_s.at[sb.at[pl.ds(0, ECHUNK)]],
                                 rbufs[0], gsems[0])
        for j in range(SUP):
            gd[j].wait()
            sd[j] = pltpu.async_copy(
                rbufs[j % 2], agg_s.at[db.at[pl.ds(j * ECHUNK, ECHUNK)]],
                ssems[j % 2], add=True)
            if j + 1 < SUP:
                if j >= 1:
                    sd[j - 1].wait()
                gd[j + 1] = pltpu.async_copy(
                    h_s.at[sb.at[pl.ds((j + 1) * ECHUNK, ECHUNK)]],
                    rbufs[(j + 1) % 2], gsems[(j + 1) % 2])
        sd[SUP - 2].wait()
        sd[SUP - 1].wait()

    @pl.loop(0, ECHUNK // 16)
    def _(v):
        onesbuf[pl.ds(v * 16, 16)] = jnp.ones((16,), jnp.float32)

    @pl.loop(0, 16)
    def _(r):
        for kk in range(DH // 16):
            zerobuf[r, pl.ds(kk * 16, 16)] = jnp.zeros((16,), jnp.float32)

    @pl.loop(0, SLAB // ECHUNK)
    def _(p):
        pltpu.sync_copy(onesbuf, deg_s.at[pl.ds(slab + p * ECHUNK, ECHUNK)])

    plsc.subcore_barrier()

    def dhalf(par):
        db = dbufs[par]
        sd = [None] * SUP
        for j in range(SUP):
            if j >= 2:
                sd[j - 2].wait()
            sd[j] = pltpu.async_copy(
                onesbuf, deg_s.at[db.at[pl.ds(j * ECHUNK, ECHUNK)]],
                ssems[j % 2], add=True)
        sd[SUP - 2].wait()
        sd[SUP - 1].wait()

    issue_idx(0, 0)

    @pl.loop(0, NSUP // 2)
    def _(gi):
        s0 = 2 * gi
        wait_idx(0)
        issue_idx(s0 + 1, 1)
        dhalf(0)
        wait_idx(1)

        @pl.when(s0 + 2 < NSUP)
        def _():
            issue_idx(s0 + 2, 0)

        dhalf(1)

    plsc.subcore_barrier()

    pltpu.sync_copy(deg_s.at[pl.ds(slab, SLAB)], disbuf.at[pl.ds(0, SLAB)])

    @pl.loop(0, SLAB // 16)
    def _(v):
        d = disbuf[pl.ds(v * 16, 16)]
        i = lax.bitcast_convert_type(d, jnp.int32)
        i = jnp.int32(0x5F3759DF) - (i >> 1)
        y = lax.bitcast_convert_type(i, jnp.float32)
        for _ in range(4):
            y = y * (1.5 - 0.5 * d * y * y)
        disbuf[pl.ds(v * 16, 16)] = y

    @pl.loop(0, SLAB // PCH)
    def _(ch):
        base_l = ch * PCH
        pltpu.sync_copy(h0_hbm.at[c, pl.ds(slab + base_l, PCH), :], h0buf)

        @pl.loop(0, PCH)
        def _(i):
            li = base_l + i
            dv = disbuf[pl.ds(li, 16)]
            dspl = jnp.full((16,), dv[0], jnp.float32)
            for kk in range(DH // 16):
                sl = pl.ds(kk * 16, 16)
                hscbuf[i, sl] = h0buf[i, sl] * dspl

        pltpu.sync_copy(hscbuf, h_s.at[pl.ds(slab + base_l, PCH), :])
        for z in range(PCH // 16):
            pltpu.sync_copy(zerobuf,
                            agg_s.at[pl.ds(slab + base_l + z * 16, 16), :])

    plsc.subcore_barrier()

    issue_idx(0, 0)

    @pl.loop(0, K_ITERS)
    def _(k):
        @pl.loop(0, NSUP // 2)
        def _(gi):
            s0 = 2 * gi
            wait_idx(0)
            issue_idx(s0 + 1, 1)
            half(0)
            wait_idx(1)

            @pl.when(s0 + 2 < NSUP)
            def _():
                issue_idx(s0 + 2, 0)

            half(1)

        @pl.when(k < K_ITERS - 1)
        def _():
            issue_idx(0, 0)

        plsc.subcore_barrier()

        issue_pw(0, 0)

        @pl.loop(0, NPCH // 2)
        def _(pp):
            for par in range(2):
                ch = 2 * pp + par
                base_l = ch * PCH
                gbase = slab + base_l
                wait_pw(par)
                if par == 0:
                    issue_pw(ch + 1, 1)
                else:
                    @pl.when(ch + 1 < NPCH)
                    def _():
                        issue_pw(ch + 1, 0)
                ab, hb, ob = pwa[par], pwh[par], pwo[par]
                for z in range(PCH // 16):
                    pltpu.sync_copy(zerobuf,
                                    agg_s.at[pl.ds(gbase + z * 16, 16), :])

                @pl.loop(0, PCH)
                def _(i):
                    li = base_l + i
                    dv = disbuf[pl.ds(li, 16)]
                    dspl = jnp.full((16,), dv[0], jnp.float32)
                    d1 = dspl * (1.0 - ALPHA)
                    for kk in range(DH // 16):
                        sl = pl.ds(kk * 16, 16)
                        hn = (d1 * (ab[i, sl] + hb[i, sl])
                              + ALPHA * ob[i, sl])
                        ab[i, sl] = hn
                        hb[i, sl] = hn * dspl

                @pl.when(k < K_ITERS - 1)
                def _():
                    pltpu.sync_copy(hb, h_s.at[pl.ds(gbase, PCH), :])

                @pl.when(k == K_ITERS - 1)
                def _():
                    pltpu.sync_copy(ab, out_hbm.at[c, pl.ds(gbase, PCH), :])

        plsc.subcore_barrier()


def kernel(x, edge_index, W1, b1, gamma, beta, W2, b2):
    h0p = pl.pallas_call(
        _mlp_body,
        out_shape=jax.ShapeDtypeStruct((2, NPAD, DH), jnp.float32),
    )(x, W1, b1.reshape(1, D), gamma.reshape(1, D), beta.reshape(1, D),
      W2, b2.reshape(1, D))

    src = edge_index[0].astype(jnp.int32)
    dst = edge_index[1].astype(jnp.int32)
    pad = N + (jnp.arange(EPAD - E, dtype=jnp.int32) % (NPAD - N))
    src2 = jnp.concatenate([src, pad])
    dst2 = jnp.concatenate([dst, pad])

    embp = _propagate(h0p, src2, dst2)

    out, emb = pl.pallas_call(
        _lsm_body,
        out_shape=[jax.ShapeDtypeStruct((N, D), jnp.float32),
                   jax.ShapeDtypeStruct((N, D), jnp.float32)],
    )(embp)
    return (out, emb)
